# baseline (speedup 1.0000x reference)
import functools

import jax
import jax.numpy as jnp
from jax import lax
from jax.experimental import pallas as pl
from jax.experimental.pallas import tpu as pltpu
from jax.experimental.pallas import tpu_sc as plsc

NODE_NUM = 1000000
HIDDEN_DIM = 64
BATCH = 16384
_PAD = 128

_info = plsc.get_sparse_core_info()
_NC, _NS = _info.num_cores, _info.num_subcores
_NW = _NC * _NS
_B_PER_W = BATCH // _NW


@functools.partial(
    pl.kernel,
    mesh=plsc.VectorSubcoreMesh(core_axis_name="c", subcore_axis_name="s"),
    out_type=jax.ShapeDtypeStruct((BATCH, _PAD), jnp.float32),
    compiler_params=pltpu.CompilerParams(use_tc_tiling_on_sc=False),
    scratch_types=[
        pltpu.VMEM((_B_PER_W,), jnp.int32),
        pltpu.VMEM((_B_PER_W, HIDDEN_DIM), jnp.float32),
        pltpu.SemaphoreType.DMA,
    ],
)
def _gather_kernel(idx_hbm, table_hbm, out_hbm, idx_v, rows_v, sem):
    wid = lax.axis_index("s") * _NC + lax.axis_index("c")
    base = wid * _B_PER_W
    pltpu.sync_copy(idx_hbm.at[pl.ds(base, _B_PER_W)], idx_v)
    pltpu.async_copy(table_hbm.at[idx_v], rows_v, sem).wait()
    pltpu.sync_copy(
        rows_v, out_hbm.at[pl.ds(base, _B_PER_W), pl.ds(0, HIDDEN_DIM)]
    )


def kernel(indices, table):
    padded = _gather_kernel(indices.astype(jnp.int32), table)
    return padded[:, :HIDDEN_DIM]


# speedup vs baseline: 1.7027x; 1.7027x over previous
import functools

import jax
import jax.numpy as jnp
from jax import lax
from jax.experimental import pallas as pl
from jax.experimental.pallas import tpu as pltpu
from jax.experimental.pallas import tpu_sc as plsc

NODE_NUM = 1000000
HIDDEN_DIM = 64
BATCH = 16384

_info = plsc.get_sparse_core_info()
_NC, _NS = _info.num_cores, _info.num_subcores
_NW = _NC * _NS
_B_PER_W = BATCH // _NW
_CHUNK = 16


@functools.partial(
    pl.kernel,
    mesh=plsc.VectorSubcoreMesh(core_axis_name="c", subcore_axis_name="s"),
    out_type=jax.ShapeDtypeStruct((BATCH, HIDDEN_DIM), jnp.float32),
    scratch_types=[
        pltpu.VMEM((_B_PER_W,), jnp.int32),
        pltpu.VMEM((_B_PER_W, HIDDEN_DIM), jnp.float32),
        pltpu.SemaphoreType.DMA,
        pltpu.SemaphoreType.DMA,
    ],
)
def _gather_kernel(idx_hbm, table_hbm, out_hbm, idx_v, rows_v, sem0, sem1):
    wid = lax.axis_index("s") * _NC + lax.axis_index("c")
    base = wid * _B_PER_W
    sems = (sem0, sem1)
    pltpu.sync_copy(idx_hbm.at[pl.ds(base, _B_PER_W)], idx_v)

    @pl.loop(0, _B_PER_W // _CHUNK)
    def _fire(i):
        v = idx_v[pl.ds(i * _CHUNK, _CHUNK)]
        for t in range(_CHUNK):
            r = v[t]
            pltpu.make_async_copy(
                table_hbm.at[pl.ds(r, 1), :],
                rows_v.at[pl.ds(i * _CHUNK + t, 1), :],
                sems[t % 2],
            ).start()

    pltpu.make_async_copy(
        table_hbm.at[pl.ds(0, _B_PER_W // 2), :],
        rows_v.at[pl.ds(0, _B_PER_W // 2), :],
        sem0,
    ).wait()
    pltpu.make_async_copy(
        table_hbm.at[pl.ds(0, _B_PER_W // 2), :],
        rows_v.at[pl.ds(0, _B_PER_W // 2), :],
        sem1,
    ).wait()
    pltpu.sync_copy(rows_v, out_hbm.at[pl.ds(base, _B_PER_W)])


def kernel(indices, table):
    return _gather_kernel(indices.astype(jnp.int32), table)
